# baseline (speedup 1.0000x reference)
import functools

import jax
import jax.numpy as jnp
from jax import lax
from jax.experimental import pallas as pl
from jax.experimental.pallas import tpu as pltpu
from jax.experimental.pallas import tpu_sc as plsc

_NUM_CORES = 2
_NUM_SUBCORES = 16
_NW = _NUM_CORES * _NUM_SUBCORES
_LANES = 16

_CHUNK = 1024
_GRP = 128


def _transform_table(data_table, W, b2, unk, blk):
    V, D = data_table.shape
    O = W.shape[0]
    grid = V // blk

    def body(x_ref, w_ref, b_ref, unk_ref, out_ref):
        t = lax.dot_general(
            x_ref[...], w_ref[...],
            (((1,), (1,)), ((), ())),
            preferred_element_type=jnp.float32,
        )
        out_ref[...] = t + b_ref[...]

        @pl.when(pl.program_id(0) == grid - 1)
        def _():
            out_ref[blk - 1, :] = unk_ref[0, :]

    return pl.pallas_call(
        body,
        grid=(grid,),
        in_specs=[
            pl.BlockSpec((blk, D), lambda i: (i, 0)),
            pl.BlockSpec((O, D), lambda i: (0, 0)),
            pl.BlockSpec((1, O), lambda i: (0, 0)),
            pl.BlockSpec((1, O), lambda i: (0, 0)),
        ],
        out_specs=pl.BlockSpec((blk, O), lambda i: (i, 0)),
        out_shape=jax.ShapeDtypeStruct((V, O), jnp.float32),
    )(data_table, W, b2, unk)


@functools.lru_cache(maxsize=None)
def _make_gather(V, O, N):
    per_w = N // _NW
    n_chunks = per_w // _CHUNK
    mesh = plsc.VectorSubcoreMesh(core_axis_name="c", subcore_axis_name="s")

    @functools.partial(
        pl.kernel,
        mesh=mesh,
        compiler_params=pltpu.CompilerParams(use_tc_tiling_on_sc=False),
        out_type=jax.ShapeDtypeStruct((N, O), jnp.float32),
        scratch_types=[
            pltpu.VMEM((_CHUNK,), jnp.int32),
            pltpu.VMEM((_CHUNK // _GRP, _GRP), jnp.int32),
            pltpu.VMEM((_CHUNK, O), jnp.float32),
            pltpu.SemaphoreType.DMA,
        ],
    )
    def gather_kernel(tbl_hbm, idx_hbm, out_hbm, idx_raw, idx_map, rows, sem):
        wid = lax.axis_index("s") * _NUM_CORES + lax.axis_index("c")
        base0 = wid * per_w

        def chunk_body(ci, carry):
            base = base0 + ci * _CHUNK
            pltpu.sync_copy(idx_hbm.at[pl.ds(base, _CHUNK)], idx_raw)
            for i in range(_CHUNK // _LANES):
                v = idx_raw[pl.ds(i * _LANES, _LANES)]
                j, k = divmod(i * _LANES, _GRP)
                idx_map[j, pl.ds(k, _LANES)] = jnp.where(v < 1, V - 1, v - 1)
            copies = [
                pltpu.async_copy(
                    tbl_hbm.at[idx_map.at[g]],
                    rows.at[pl.ds(g * _GRP, _GRP)],
                    sem,
                )
                for g in range(_CHUNK // _GRP)
            ]
            for c in copies:
                c.wait()
            pltpu.sync_copy(rows, out_hbm.at[pl.ds(base, _CHUNK)])
            return carry

        lax.fori_loop(0, n_chunks, chunk_body, 0)

    return gather_kernel


def _finalize(G, B, H, O, nbf):
    grid = B // nbf

    def body(g_ref, out_ref):
        out_ref[...] = g_ref[...].reshape(nbf, H, O)

    return pl.pallas_call(
        body,
        grid=(grid,),
        in_specs=[pl.BlockSpec((nbf * H, O), lambda i: (i, 0))],
        out_specs=pl.BlockSpec((nbf, H, O), lambda i: (i, 0, 0)),
        out_shape=jax.ShapeDtypeStruct((B, H, O), jnp.float32),
    )(G)


def kernel(indices, data_table, unknown_table, W, b):
    V, D = data_table.shape
    O = W.shape[0]
    B, H = indices.shape
    N = B * H

    T = _transform_table(
        data_table, W, b.reshape(1, O), unknown_table, blk=5000
    )
    idx_flat = indices.reshape(N).astype(jnp.int32)
    G = _make_gather(V, O, N)(T, idx_flat)
    return _finalize(G, B, H, O, nbf=64)


# speedup vs baseline: 1.2851x; 1.2851x over previous
import functools

import jax
import jax.numpy as jnp
from jax import lax
from jax.experimental import pallas as pl
from jax.experimental.pallas import tpu as pltpu
from jax.experimental.pallas import tpu_sc as plsc

_NUM_CORES = 2
_NUM_SUBCORES = 16
_NW = _NUM_CORES * _NUM_SUBCORES
_LANES = 16

_CHUNK = 512
_GRP = 128


def _transform_table(data_table, W, b2, unk, blk):
    V, D = data_table.shape
    O = W.shape[0]
    grid = V // blk

    def body(x_ref, w_ref, b_ref, unk_ref, out_ref):
        t = lax.dot_general(
            x_ref[...], w_ref[...],
            (((1,), (1,)), ((), ())),
            preferred_element_type=jnp.float32,
        )
        out_ref[...] = t + b_ref[...]

        @pl.when(pl.program_id(0) == grid - 1)
        def _():
            out_ref[blk - 1, :] = unk_ref[0, :]

    return pl.pallas_call(
        body,
        grid=(grid,),
        in_specs=[
            pl.BlockSpec((blk, D), lambda i: (i, 0)),
            pl.BlockSpec((O, D), lambda i: (0, 0)),
            pl.BlockSpec((1, O), lambda i: (0, 0)),
            pl.BlockSpec((1, O), lambda i: (0, 0)),
        ],
        out_specs=pl.BlockSpec((blk, O), lambda i: (i, 0)),
        out_shape=jax.ShapeDtypeStruct((V, O), jnp.float32),
    )(data_table, W, b2, unk)


@functools.lru_cache(maxsize=None)
def _make_gather(V, O, N):
    M = N // 2
    per_w = M // _NW
    n_chunks = per_w // _CHUNK
    mesh = plsc.VectorSubcoreMesh(core_axis_name="c", subcore_axis_name="s")

    @functools.partial(
        pl.kernel,
        mesh=mesh,
        compiler_params=pltpu.CompilerParams(use_tc_tiling_on_sc=False),
        out_type=jax.ShapeDtypeStruct((M, 2 * O), jnp.float32),
        scratch_types=[
            pltpu.VMEM((_CHUNK,), jnp.int32),
            pltpu.VMEM((_CHUNK,), jnp.int32),
            pltpu.VMEM((_CHUNK // _GRP, _GRP), jnp.int32),
            pltpu.VMEM((_CHUNK // _GRP, _GRP), jnp.int32),
            pltpu.VMEM((_CHUNK, O), jnp.float32),
            pltpu.VMEM((_CHUNK, O), jnp.float32),
            pltpu.SemaphoreType.DMA,
        ],
    )
    def gather_kernel(tbl_hbm, idx_hbm, out_hbm, idx_l, idx_r,
                      map_l, map_r, rows_l, rows_r, sem):
        wid = lax.axis_index("s") * _NUM_CORES + lax.axis_index("c")
        base0 = wid * per_w

        def chunk_body(ci, carry):
            base = base0 + ci * _CHUNK
            pltpu.sync_copy(idx_hbm.at[pl.ds(base, _CHUNK)], idx_l)
            pltpu.sync_copy(idx_hbm.at[pl.ds(M + base, _CHUNK)], idx_r)
            for src, dst in ((idx_l, map_l), (idx_r, map_r)):
                for i in range(_CHUNK // _LANES):
                    v = src[pl.ds(i * _LANES, _LANES)]
                    j, k = divmod(i * _LANES, _GRP)
                    dst[j, pl.ds(k, _LANES)] = jnp.where(v < 1, V - 1, v - 1)
            copies = [
                pltpu.async_copy(
                    tbl_hbm.at[m.at[g]],
                    r.at[pl.ds(g * _GRP, _GRP)],
                    sem,
                )
                for g in range(_CHUNK // _GRP)
                for m, r in ((map_l, rows_l), (map_r, rows_r))
            ]
            for c in copies:
                c.wait()
            pltpu.sync_copy(
                rows_l, out_hbm.at[pl.ds(base, _CHUNK), pl.ds(0, O)])
            pltpu.sync_copy(
                rows_r, out_hbm.at[pl.ds(base, _CHUNK), pl.ds(O, O)])
            return carry

        lax.fori_loop(0, n_chunks, chunk_body, 0)

    return gather_kernel


def _finalize(G2, B, H, O, nbf):
    grid = B // nbf
    half = grid // 2

    def body(g_ref, out_ref):
        x = g_ref[...]
        half_sel = jnp.where(pl.program_id(0) < half, x[:, :O], x[:, O:])
        out_ref[...] = half_sel.reshape(nbf, H, O)

    return pl.pallas_call(
        body,
        grid=(grid,),
        in_specs=[
            pl.BlockSpec((nbf * H, 2 * O), lambda i: (i % half, 0)),
        ],
        out_specs=pl.BlockSpec((nbf, H, O), lambda i: (i, 0, 0)),
        out_shape=jax.ShapeDtypeStruct((B, H, O), jnp.float32),
    )(G2)


def kernel(indices, data_table, unknown_table, W, b):
    V, D = data_table.shape
    O = W.shape[0]
    B, H = indices.shape
    N = B * H

    T = _transform_table(
        data_table, W, b.reshape(1, O), unknown_table, blk=5000
    )
    idx_flat = indices.reshape(N).astype(jnp.int32)
    G2 = _make_gather(V, O, N)(T, idx_flat)
    return _finalize(G2, B, H, O, nbf=64)


# speedup vs baseline: 2.4132x; 1.8779x over previous
import functools

import jax
import jax.numpy as jnp
from jax import lax
from jax.experimental import pallas as pl
from jax.experimental.pallas import tpu as pltpu
from jax.experimental.pallas import tpu_sc as plsc

_NUM_CORES = 2
_NUM_SUBCORES = 16
_NW = _NUM_CORES * _NUM_SUBCORES
_LANES = 16

_CHUNK = 512
_GRP = 128


def _transform_table(data_table, W, b2, unk, blk):
    V, D = data_table.shape
    O = W.shape[0]
    grid = V // blk

    def body(x_ref, w_ref, b_ref, unk_ref, out_ref):
        t = lax.dot_general(
            x_ref[...], w_ref[...],
            (((1,), (1,)), ((), ())),
            preferred_element_type=jnp.float32,
        )
        out_ref[...] = t + b_ref[...]

        @pl.when(pl.program_id(0) == grid - 1)
        def _():
            out_ref[blk - 1, :] = unk_ref[0, :]

    return pl.pallas_call(
        body,
        grid=(grid,),
        in_specs=[
            pl.BlockSpec((blk, D), lambda i: (i, 0)),
            pl.BlockSpec((O, D), lambda i: (0, 0)),
            pl.BlockSpec((1, O), lambda i: (0, 0)),
            pl.BlockSpec((1, O), lambda i: (0, 0)),
        ],
        out_specs=pl.BlockSpec((blk, O), lambda i: (i, 0)),
        out_shape=jax.ShapeDtypeStruct((V, O), jnp.float32),
    )(data_table, W, b2, unk)


@functools.lru_cache(maxsize=None)
def _make_gather(V, O, N):
    M = N // 2
    per_w = M // _NW
    n_chunks = per_w // _CHUNK
    mesh = plsc.VectorSubcoreMesh(core_axis_name="c", subcore_axis_name="s")

    @functools.partial(
        pl.kernel,
        mesh=mesh,
        compiler_params=pltpu.CompilerParams(use_tc_tiling_on_sc=False),
        out_type=jax.ShapeDtypeStruct((M, 2 * O), jnp.float32),
        scratch_types=[
            pltpu.VMEM((_CHUNK,), jnp.int32),
            pltpu.VMEM((_CHUNK,), jnp.int32),
            pltpu.VMEM((_CHUNK // _GRP, _GRP), jnp.int32),
            pltpu.VMEM((_CHUNK // _GRP, _GRP), jnp.int32),
            pltpu.VMEM((_CHUNK, O), jnp.float32),
            pltpu.VMEM((_CHUNK, O), jnp.float32),
            pltpu.SemaphoreType.DMA,
        ],
    )
    def gather_kernel(tbl_hbm, idx_hbm, out_hbm, idx_l, idx_r,
                      map_l, map_r, rows_l, rows_r, sem):
        wid = lax.axis_index("s") * _NUM_CORES + lax.axis_index("c")
        base0 = wid * per_w

        def chunk_body(ci, carry):
            base = base0 + ci * _CHUNK
            pltpu.sync_copy(idx_hbm.at[pl.ds(base, _CHUNK)], idx_l)
            pltpu.sync_copy(idx_hbm.at[pl.ds(M + base, _CHUNK)], idx_r)
            for src, dst in ((idx_l, map_l), (idx_r, map_r)):
                for i in range(_CHUNK // _LANES):
                    v = src[pl.ds(i * _LANES, _LANES)]
                    j, k = divmod(i * _LANES, _GRP)
                    dst[j, pl.ds(k, _LANES)] = jnp.where(v < 1, V - 1, v - 1)
            copies = [
                pltpu.async_copy(
                    tbl_hbm.at[m.at[g]],
                    r.at[pl.ds(g * _GRP, _GRP)],
                    sem,
                )
                for g in range(_CHUNK // _GRP)
                for m, r in ((map_l, rows_l), (map_r, rows_r))
            ]
            for c in copies:
                c.wait()
            pltpu.sync_copy(
                rows_l, out_hbm.at[pl.ds(base, _CHUNK), pl.ds(0, O)])
            pltpu.sync_copy(
                rows_r, out_hbm.at[pl.ds(base, _CHUNK), pl.ds(O, O)])
            return carry

        lax.fori_loop(0, n_chunks, chunk_body, 0)

    return gather_kernel


def _finalize(G3, B, H, O, nbb):
    grid = B // nbb
    HH = H // 2

    def body(g_ref, out_ref):
        x = g_ref[...]
        out_ref[0:HH] = jnp.transpose(x[:, :, :O], (0, 2, 1))
        out_ref[HH:H] = jnp.transpose(x[:, :, O:], (0, 2, 1))

    return pl.pallas_call(
        body,
        grid=(grid,),
        in_specs=[pl.BlockSpec((HH, nbb, 2 * O), lambda i: (0, i, 0))],
        out_specs=pl.BlockSpec((H, O, nbb), lambda i: (0, 0, i)),
        out_shape=jax.ShapeDtypeStruct((H, O, B), jnp.float32),
    )(G3)


def kernel(indices, data_table, unknown_table, W, b):
    V, D = data_table.shape
    O = W.shape[0]
    B, H = indices.shape
    N = B * H

    T = _transform_table(
        data_table, W, b.reshape(1, O), unknown_table, blk=5000
    )
    idx_flat = indices.T.reshape(N).astype(jnp.int32)
    G2 = _make_gather(V, O, N)(T, idx_flat)
    G3 = G2.reshape(H // 2, B, 2 * O)
    out_t = _finalize(G3, B, H, O, nbb=128)
    return jnp.transpose(out_t, (2, 0, 1))


# speedup vs baseline: 2.5735x; 1.0664x over previous
import functools

import jax
import jax.numpy as jnp
from jax import lax
from jax.experimental import pallas as pl
from jax.experimental.pallas import tpu as pltpu
from jax.experimental.pallas import tpu_sc as plsc

_NUM_CORES = 2
_NUM_SUBCORES = 16
_NW = _NUM_CORES * _NUM_SUBCORES
_LANES = 16

_CHUNK = 512
_GRP = 128


def _transform_table(data_table, W, b2, unk, blk):
    V, D = data_table.shape
    O = W.shape[0]
    grid = V // blk

    def body(x_ref, w_ref, b_ref, unk_ref, out_ref):
        t = lax.dot_general(
            x_ref[...], w_ref[...],
            (((1,), (1,)), ((), ())),
            preferred_element_type=jnp.float32,
        )
        out_ref[...] = t + b_ref[...]

        @pl.when(pl.program_id(0) == grid - 1)
        def _():
            out_ref[blk - 1, :] = unk_ref[0, :]

    return pl.pallas_call(
        body,
        grid=(grid,),
        in_specs=[
            pl.BlockSpec((blk, D), lambda i: (i, 0)),
            pl.BlockSpec((O, D), lambda i: (0, 0)),
            pl.BlockSpec((1, O), lambda i: (0, 0)),
            pl.BlockSpec((1, O), lambda i: (0, 0)),
        ],
        out_specs=pl.BlockSpec((blk, O), lambda i: (i, 0)),
        out_shape=jax.ShapeDtypeStruct((V, O), jnp.float32),
    )(data_table, W, b2, unk)


@functools.lru_cache(maxsize=None)
def _make_gather(V, O, N):
    M = N // 2
    per_w = M // _NW
    n_chunks = per_w // _CHUNK
    mesh = plsc.VectorSubcoreMesh(core_axis_name="c", subcore_axis_name="s")

    @functools.partial(
        pl.kernel,
        mesh=mesh,
        compiler_params=pltpu.CompilerParams(use_tc_tiling_on_sc=False),
        out_type=jax.ShapeDtypeStruct((M, 2 * O), jnp.float32),
        scratch_types=[
            pltpu.VMEM((_CHUNK,), jnp.int32),
            pltpu.VMEM((_CHUNK,), jnp.int32),
            pltpu.VMEM((_CHUNK // _GRP, _GRP), jnp.int32),
            pltpu.VMEM((_CHUNK // _GRP, _GRP), jnp.int32),
            pltpu.VMEM((_CHUNK, O), jnp.float32),
            pltpu.VMEM((_CHUNK, O), jnp.float32),
            pltpu.SemaphoreType.DMA,
        ],
    )
    def gather_kernel(tbl_hbm, idx_hbm, out_hbm, idx_l, idx_r,
                      map_l, map_r, rows_l, rows_r, sem):
        wid = lax.axis_index("s") * _NUM_CORES + lax.axis_index("c")
        base0 = wid * per_w

        def chunk_body(ci, carry):
            base = base0 + ci * _CHUNK
            pltpu.sync_copy(idx_hbm.at[pl.ds(base, _CHUNK)], idx_l)
            pltpu.sync_copy(idx_hbm.at[pl.ds(M + base, _CHUNK)], idx_r)
            for src, dst in ((idx_l, map_l), (idx_r, map_r)):
                for i in range(_CHUNK // _LANES):
                    v = src[pl.ds(i * _LANES, _LANES)]
                    j, k = divmod(i * _LANES, _GRP)
                    dst[j, pl.ds(k, _LANES)] = jnp.where(v < 1, V - 1, v - 1)
            copies = [
                pltpu.async_copy(
                    tbl_hbm.at[m.at[g]],
                    r.at[pl.ds(g * _GRP, _GRP)],
                    sem,
                )
                for g in range(_CHUNK // _GRP)
                for m, r in ((map_l, rows_l), (map_r, rows_r))
            ]
            for c in copies:
                c.wait()
            pltpu.sync_copy(
                rows_l, out_hbm.at[pl.ds(base, _CHUNK), pl.ds(0, O)])
            pltpu.sync_copy(
                rows_r, out_hbm.at[pl.ds(base, _CHUNK), pl.ds(O, O)])
            return carry

        lax.fori_loop(0, n_chunks, chunk_body, 0)

    return gather_kernel


def _finalize(G3, B, H, O, nbb):
    grid = B // nbb
    HH = H // 2

    def body(g_ref, out_ref):
        x = g_ref[...]
        out_ref[0:HH] = jnp.transpose(x[:, :, :O], (0, 2, 1))
        out_ref[HH:H] = jnp.transpose(x[:, :, O:], (0, 2, 1))

    return pl.pallas_call(
        body,
        grid=(grid,),
        in_specs=[pl.BlockSpec((HH, nbb, 2 * O), lambda i: (0, i, 0))],
        out_specs=pl.BlockSpec((H, O, nbb), lambda i: (0, 0, i)),
        out_shape=jax.ShapeDtypeStruct((H, O, B), jnp.float32),
    )(G3)


def kernel(indices, data_table, unknown_table, W, b):
    V, D = data_table.shape
    O = W.shape[0]
    B, H = indices.shape
    N = B * H

    T = _transform_table(
        data_table, W, b.reshape(1, O), unknown_table, blk=5000
    )
    idx_flat = indices.T.reshape(N).astype(jnp.int32)
    G2 = _make_gather(V, O, N)(T, idx_flat)
    G3 = G2.reshape(H // 2, B, 2 * O)
    out_t = _finalize(G3, B, H, O, nbb=256)
    return jnp.transpose(out_t, (2, 0, 1))


# speedup vs baseline: 2.6760x; 1.0398x over previous
import functools

import jax
import jax.numpy as jnp
from jax import lax
from jax.experimental import pallas as pl
from jax.experimental.pallas import tpu as pltpu
from jax.experimental.pallas import tpu_sc as plsc

_NUM_CORES = 2
_NUM_SUBCORES = 16
_NW = _NUM_CORES * _NUM_SUBCORES
_LANES = 16

_CHUNK = 512
_GRP = 128


def _transform_table(data_table, W, b2, unk, blk):
    V, D = data_table.shape
    O = W.shape[0]
    grid = V // blk

    def body(x_ref, w_ref, b_ref, unk_ref, out_ref):
        t = lax.dot_general(
            x_ref[...], w_ref[...],
            (((1,), (1,)), ((), ())),
            preferred_element_type=jnp.float32,
        )
        out_ref[...] = t + b_ref[...]

        @pl.when(pl.program_id(0) == grid - 1)
        def _():
            out_ref[blk - 1, :] = unk_ref[0, :]

    return pl.pallas_call(
        body,
        grid=(grid,),
        in_specs=[
            pl.BlockSpec((blk, D), lambda i: (i, 0)),
            pl.BlockSpec((O, D), lambda i: (0, 0)),
            pl.BlockSpec((1, O), lambda i: (0, 0)),
            pl.BlockSpec((1, O), lambda i: (0, 0)),
        ],
        out_specs=pl.BlockSpec((blk, O), lambda i: (i, 0)),
        out_shape=jax.ShapeDtypeStruct((V, O), jnp.float32),
    )(data_table, W, b2, unk)


@functools.lru_cache(maxsize=None)
def _make_gather(V, O, N):
    M = N // 2
    per_w = M // _NW
    n_chunks = per_w // _CHUNK
    mesh = plsc.VectorSubcoreMesh(core_axis_name="c", subcore_axis_name="s")

    @functools.partial(
        pl.kernel,
        mesh=mesh,
        compiler_params=pltpu.CompilerParams(use_tc_tiling_on_sc=False),
        out_type=jax.ShapeDtypeStruct((M, 2 * O), jnp.float32),
        scratch_types=[
            pltpu.VMEM((_CHUNK,), jnp.int32),
            pltpu.VMEM((_CHUNK,), jnp.int32),
            pltpu.VMEM((_CHUNK // _GRP, _GRP), jnp.int32),
            pltpu.VMEM((_CHUNK // _GRP, _GRP), jnp.int32),
            pltpu.VMEM((_CHUNK, O), jnp.float32),
            pltpu.VMEM((_CHUNK, O), jnp.float32),
            pltpu.SemaphoreType.DMA,
        ],
    )
    def gather_kernel(tbl_hbm, idx_hbm, out_hbm, idx_l, idx_r,
                      map_l, map_r, rows_l, rows_r, sem):
        wid = lax.axis_index("s") * _NUM_CORES + lax.axis_index("c")
        base0 = wid * per_w

        def chunk_body(ci, carry):
            base = base0 + ci * _CHUNK
            pltpu.sync_copy(idx_hbm.at[pl.ds(base, _CHUNK)], idx_l)
            pltpu.sync_copy(idx_hbm.at[pl.ds(M + base, _CHUNK)], idx_r)
            for src, dst in ((idx_l, map_l), (idx_r, map_r)):
                for i in range(_CHUNK // _LANES):
                    v = src[pl.ds(i * _LANES, _LANES)]
                    j, k = divmod(i * _LANES, _GRP)
                    dst[j, pl.ds(k, _LANES)] = jnp.where(v < 1, V - 1, v - 1)
            copies = [
                pltpu.async_copy(
                    tbl_hbm.at[m.at[g]],
                    r.at[pl.ds(g * _GRP, _GRP)],
                    sem,
                )
                for g in range(_CHUNK // _GRP)
                for m, r in ((map_l, rows_l), (map_r, rows_r))
            ]
            for c in copies:
                c.wait()
            pltpu.sync_copy(
                rows_l, out_hbm.at[pl.ds(base, _CHUNK), pl.ds(0, O)])
            pltpu.sync_copy(
                rows_r, out_hbm.at[pl.ds(base, _CHUNK), pl.ds(O, O)])
            return carry

        lax.fori_loop(0, n_chunks, chunk_body, 0)

    return gather_kernel


def _finalize(G3, B, H, O, nbb):
    grid = B // nbb
    HH = H // 2

    def body(g_ref, out_ref):
        x = g_ref[...]
        out_ref[0:HH] = jnp.transpose(x[:, :, :O], (0, 2, 1))
        out_ref[HH:H] = jnp.transpose(x[:, :, O:], (0, 2, 1))

    return pl.pallas_call(
        body,
        grid=(grid,),
        in_specs=[pl.BlockSpec((HH, nbb, 2 * O), lambda i: (0, i, 0))],
        out_specs=pl.BlockSpec((H, O, nbb), lambda i: (0, 0, i)),
        out_shape=jax.ShapeDtypeStruct((H, O, B), jnp.float32),
    )(G3)


def kernel(indices, data_table, unknown_table, W, b):
    V, D = data_table.shape
    O = W.shape[0]
    B, H = indices.shape
    N = B * H

    T = _transform_table(
        data_table, W, b.reshape(1, O), unknown_table, blk=5000
    )
    idx_flat = indices.T.reshape(N).astype(jnp.int32)
    G2 = _make_gather(V, O, N)(T, idx_flat)
    G3 = G2.reshape(H // 2, B, 2 * O)
    out_t = _finalize(G3, B, H, O, nbb=512)
    return jnp.transpose(out_t, (2, 0, 1))


# speedup vs baseline: 2.9280x; 1.0942x over previous
import functools

import jax
import jax.numpy as jnp
from jax import lax
from jax.experimental import pallas as pl
from jax.experimental.pallas import tpu as pltpu
from jax.experimental.pallas import tpu_sc as plsc

_NUM_CORES = 2
_NUM_SUBCORES = 16
_NW = _NUM_CORES * _NUM_SUBCORES
_LANES = 16

_CHUNK = 512
_GRP = 128


def _transform_table(data_table, W, b2, unk, blk):
    V, D = data_table.shape
    O = W.shape[0]
    grid = (V // 2) // blk

    def body(x1_ref, x2_ref, w_ref, b_ref, unk_ref, out_ref):
        dn = (((1,), (1,)), ((), ()))
        t1 = lax.dot_general(x1_ref[...], w_ref[...], dn,
                             preferred_element_type=jnp.float32)
        t2 = lax.dot_general(x2_ref[...], w_ref[...], dn,
                             preferred_element_type=jnp.float32)
        out_ref[:, 0:O] = t1 + b_ref[...]
        out_ref[:, O:2 * O] = t2 + b_ref[...]

        @pl.when(pl.program_id(0) == grid - 1)
        def _():
            out_ref[blk - 1, pl.ds(O, O)] = unk_ref[0, :]

    return pl.pallas_call(
        body,
        grid=(grid,),
        in_specs=[
            pl.BlockSpec((blk, D), lambda i: (i, 0)),
            pl.BlockSpec((blk, D), lambda i: (i + grid, 0)),
            pl.BlockSpec((O, D), lambda i: (0, 0)),
            pl.BlockSpec((1, O), lambda i: (0, 0)),
            pl.BlockSpec((1, O), lambda i: (0, 0)),
        ],
        out_specs=pl.BlockSpec((blk, 2 * O), lambda i: (i, 0)),
        out_shape=jax.ShapeDtypeStruct((V // 2, 2 * O), jnp.float32),
    )(data_table, data_table, W, b2, unk)


@functools.lru_cache(maxsize=None)
def _make_gather(V, O, N):
    M = N // 2
    per_w = M // _NW
    n_chunks = per_w // _CHUNK
    mesh = plsc.VectorSubcoreMesh(core_axis_name="c", subcore_axis_name="s")

    @functools.partial(
        pl.kernel,
        mesh=mesh,
        compiler_params=pltpu.CompilerParams(use_tc_tiling_on_sc=False),
        out_type=jax.ShapeDtypeStruct((M, 2 * O), jnp.float32),
        scratch_types=[
            pltpu.VMEM((_CHUNK,), jnp.int32),
            pltpu.VMEM((_CHUNK,), jnp.int32),
            pltpu.VMEM((_CHUNK // _GRP, _GRP), jnp.int32),
            pltpu.VMEM((_CHUNK // _GRP, _GRP), jnp.int32),
            pltpu.VMEM((_CHUNK, O), jnp.float32),
            pltpu.VMEM((_CHUNK, O), jnp.float32),
            pltpu.SemaphoreType.DMA,
        ],
    )
    def gather_kernel(tbl_hbm, idx_hbm, out_hbm, idx_l, idx_r,
                      map_l, map_r, rows_l, rows_r, sem):
        wid = lax.axis_index("s") * _NUM_CORES + lax.axis_index("c")
        base0 = wid * per_w

        def chunk_body(ci, carry):
            base = base0 + ci * _CHUNK
            pltpu.sync_copy(idx_hbm.at[pl.ds(base, _CHUNK)], idx_l)
            pltpu.sync_copy(idx_hbm.at[pl.ds(M + base, _CHUNK)], idx_r)
            for src, dst in ((idx_l, map_l), (idx_r, map_r)):
                for i in range(_CHUNK // _LANES):
                    v = src[pl.ds(i * _LANES, _LANES)]
                    j, k = divmod(i * _LANES, _GRP)
                    r = jnp.where(v < 1, V - 1, v - 1)
                    dst[j, pl.ds(k, _LANES)] = (
                        jnp.where(r < V // 2, r + r, r + r - (V - 1)))
            copies = [
                pltpu.async_copy(
                    tbl_hbm.at[m.at[g]],
                    r.at[pl.ds(g * _GRP, _GRP)],
                    sem,
                )
                for g in range(_CHUNK // _GRP)
                for m, r in ((map_l, rows_l), (map_r, rows_r))
            ]
            for c in copies:
                c.wait()
            pltpu.sync_copy(
                rows_l, out_hbm.at[pl.ds(base, _CHUNK), pl.ds(0, O)])
            pltpu.sync_copy(
                rows_r, out_hbm.at[pl.ds(base, _CHUNK), pl.ds(O, O)])
            return carry

        lax.fori_loop(0, n_chunks, chunk_body, 0)

    return gather_kernel


def _finalize(G3, B, H, O, nbb):
    grid = B // nbb
    HH = H // 2

    def body(g_ref, out_ref):
        x = g_ref[...]
        out_ref[0:HH] = jnp.transpose(x[:, :, :O], (0, 2, 1))
        out_ref[HH:H] = jnp.transpose(x[:, :, O:], (0, 2, 1))

    return pl.pallas_call(
        body,
        grid=(grid,),
        in_specs=[pl.BlockSpec((HH, nbb, 2 * O), lambda i: (0, i, 0))],
        out_specs=pl.BlockSpec((H, O, nbb), lambda i: (0, 0, i)),
        out_shape=jax.ShapeDtypeStruct((H, O, B), jnp.float32),
    )(G3)


def kernel(indices, data_table, unknown_table, W, b):
    V, D = data_table.shape
    O = W.shape[0]
    B, H = indices.shape
    N = B * H

    T2 = _transform_table(
        data_table, W, b.reshape(1, O), unknown_table, blk=2000
    )
    idx_flat = indices.T.reshape(N).astype(jnp.int32)
    G2 = _make_gather(V, O, N)(T2.reshape(V, O), idx_flat)
    G3 = G2.reshape(H // 2, B, 2 * O)
    out_t = _finalize(G3, B, H, O, nbb=512)
    return jnp.transpose(out_t, (2, 0, 1))


# speedup vs baseline: 3.0765x; 1.0507x over previous
import functools

import jax
import jax.numpy as jnp
from jax import lax
from jax.experimental import pallas as pl
from jax.experimental.pallas import tpu as pltpu
from jax.experimental.pallas import tpu_sc as plsc

_NUM_CORES = 2
_NUM_SUBCORES = 16
_NW = _NUM_CORES * _NUM_SUBCORES
_LANES = 16

_CHUNK = 512
_GRP = 128


def _transform_table(data_table, W, b2, unk, blk):
    V, D = data_table.shape
    O = W.shape[0]
    grid = (V // 2) // blk

    def body(x1_ref, x2_ref, w_ref, b_ref, unk_ref, out_ref):
        dn = (((1,), (1,)), ((), ()))
        t1 = lax.dot_general(x1_ref[...], w_ref[...], dn,
                             preferred_element_type=jnp.float32)
        t2 = lax.dot_general(x2_ref[...], w_ref[...], dn,
                             preferred_element_type=jnp.float32)
        out_ref[:, 0:O] = t1 + b_ref[...]
        out_ref[:, O:2 * O] = t2 + b_ref[...]

        @pl.when(pl.program_id(0) == grid - 1)
        def _():
            out_ref[blk - 1, pl.ds(O, O)] = unk_ref[0, :]

    return pl.pallas_call(
        body,
        grid=(grid,),
        in_specs=[
            pl.BlockSpec((blk, D), lambda i: (i, 0)),
            pl.BlockSpec((blk, D), lambda i: (i + grid, 0)),
            pl.BlockSpec((O, D), lambda i: (0, 0)),
            pl.BlockSpec((1, O), lambda i: (0, 0)),
            pl.BlockSpec((1, O), lambda i: (0, 0)),
        ],
        out_specs=pl.BlockSpec((blk, 2 * O), lambda i: (i, 0)),
        out_shape=jax.ShapeDtypeStruct((V // 2, 2 * O), jnp.float32),
    )(data_table, data_table, W, b2, unk)


@functools.lru_cache(maxsize=None)
def _make_gather(V, O, N):
    M = N // 2
    per_w = M // _NW
    n_chunks = per_w // _CHUNK
    mesh = plsc.VectorSubcoreMesh(core_axis_name="c", subcore_axis_name="s")

    @functools.partial(
        pl.kernel,
        mesh=mesh,
        compiler_params=pltpu.CompilerParams(use_tc_tiling_on_sc=False),
        out_type=jax.ShapeDtypeStruct((M, 2 * O), jnp.float32),
        scratch_types=[
            pltpu.VMEM((_CHUNK,), jnp.int32),
            pltpu.VMEM((_CHUNK,), jnp.int32),
            pltpu.VMEM((_CHUNK // _GRP, _GRP), jnp.int32),
            pltpu.VMEM((_CHUNK // _GRP, _GRP), jnp.int32),
            pltpu.VMEM((_CHUNK, O), jnp.float32),
            pltpu.VMEM((_CHUNK, O), jnp.float32),
            pltpu.SemaphoreType.DMA,
            pltpu.SemaphoreType.DMA,
        ],
    )
    def gather_kernel(tbl_hbm, idx_hbm, out_hbm, idx_l, idx_r,
                      map_l, map_r, rows_l, rows_r, sem, wsem):
        wid = lax.axis_index("s") * _NUM_CORES + lax.axis_index("c")
        base0 = wid * per_w

        def chunk_body(ci, carry):
            base = base0 + ci * _CHUNK
            pltpu.sync_copy(idx_hbm.at[pl.ds(base, _CHUNK)], idx_l)
            pltpu.sync_copy(idx_hbm.at[pl.ds(M + base, _CHUNK)], idx_r)
            for src, dst in ((idx_l, map_l), (idx_r, map_r)):
                for i in range(_CHUNK // _LANES):
                    v = src[pl.ds(i * _LANES, _LANES)]
                    j, k = divmod(i * _LANES, _GRP)
                    r = jnp.where(v < 1, V - 1, v - 1)
                    dst[j, pl.ds(k, _LANES)] = (
                        jnp.where(r < V // 2, r + r, r + r - (V - 1)))
            @pl.when(ci > 0)
            def _():
                pltpu.make_async_copy(
                    rows_l, out_hbm.at[pl.ds(base, _CHUNK), pl.ds(0, O)],
                    wsem).wait()
                pltpu.make_async_copy(
                    rows_r, out_hbm.at[pl.ds(base, _CHUNK), pl.ds(O, O)],
                    wsem).wait()
            copies = [
                pltpu.async_copy(
                    tbl_hbm.at[m.at[g]],
                    r.at[pl.ds(g * _GRP, _GRP)],
                    sem,
                )
                for g in range(_CHUNK // _GRP)
                for m, r in ((map_l, rows_l), (map_r, rows_r))
            ]
            for c in copies:
                c.wait()
            pltpu.async_copy(
                rows_l, out_hbm.at[pl.ds(base, _CHUNK), pl.ds(0, O)], wsem)
            pltpu.async_copy(
                rows_r, out_hbm.at[pl.ds(base, _CHUNK), pl.ds(O, O)], wsem)
            return carry

        lax.fori_loop(0, n_chunks, chunk_body, 0)
        last = base0 + (n_chunks - 1) * _CHUNK
        pltpu.make_async_copy(
            rows_l, out_hbm.at[pl.ds(last, _CHUNK), pl.ds(0, O)],
            wsem).wait()
        pltpu.make_async_copy(
            rows_r, out_hbm.at[pl.ds(last, _CHUNK), pl.ds(O, O)],
            wsem).wait()

    return gather_kernel


def _finalize(G3, B, H, O, nbb):
    grid = B // nbb
    HH = H // 2

    def body(g_ref, out_ref):
        x = g_ref[...]
        out_ref[0:HH] = jnp.transpose(x[:, :, :O], (0, 2, 1))
        out_ref[HH:H] = jnp.transpose(x[:, :, O:], (0, 2, 1))

    return pl.pallas_call(
        body,
        grid=(grid,),
        in_specs=[pl.BlockSpec((HH, nbb, 2 * O), lambda i: (0, i, 0))],
        out_specs=pl.BlockSpec((H, O, nbb), lambda i: (0, 0, i)),
        out_shape=jax.ShapeDtypeStruct((H, O, B), jnp.float32),
    )(G3)


def kernel(indices, data_table, unknown_table, W, b):
    V, D = data_table.shape
    O = W.shape[0]
    B, H = indices.shape
    N = B * H

    T2 = _transform_table(
        data_table, W, b.reshape(1, O), unknown_table, blk=2000
    )
    idx_flat = indices.T.reshape(N).astype(jnp.int32)
    G2 = _make_gather(V, O, N)(T2.reshape(V, O), idx_flat)
    G3 = G2.reshape(H // 2, B, 2 * O)
    out_t = _finalize(G3, B, H, O, nbb=512)
    return jnp.transpose(out_t, (2, 0, 1))
